# baseline (speedup 1.0000x reference)
import functools

import jax
import jax.numpy as jnp
from jax.experimental import pallas as pl
from jax.experimental.pallas import tpu as pltpu

K = 2


def _normalize(v, eps=1e-08):
    n = jnp.sqrt(jnp.sum(v * v, axis=-1, keepdims=True))
    return v / jnp.maximum(n, eps)


def _ffn_kernel(x_ref, w_ref, W1_ref, b1_ref, W2_ref, b2_ref, o_ref, acc_ref):
    e = pl.program_id(1)
    f = pl.program_id(2)
    nE = pl.num_programs(1)
    nF = pl.num_programs(2)

    @pl.when((e == 0) & (f == 0))
    def _():
        acc_ref[...] = jnp.zeros_like(acc_ref)

    h = jax.nn.gelu(
        jax.lax.dot_general(
            x_ref[...], W1_ref[0],
            (((1,), (0,)), ((), ())),
            preferred_element_type=jnp.float32,
        )
        + b1_ref[0, 0]
    )
    oe = jax.lax.dot_general(
        h, W2_ref[0],
        (((1,), (0,)), ((), ())),
        preferred_element_type=jnp.float32,
    )
    lane = jax.lax.broadcasted_iota(jnp.int32, w_ref.shape, 1)
    wcol = jnp.sum(jnp.where(lane == e, w_ref[...], 0.0), axis=1)

    @pl.when(f == 0)
    def _():
        acc_ref[...] += b2_ref[0, 0] * wcol[:, None]

    acc_ref[...] += oe * wcol[:, None]

    @pl.when((e == nE - 1) & (f == nF - 1))
    def _():
        o_ref[...] = acc_ref[...]


def kernel(x, anchors, W1, b1, W2, b2):
    Bq, Sq, Dq = x.shape
    E, _, DFF = W1.shape
    T = Bq * Sq
    x_flat = x.reshape(T, Dq)

    x_norm = _normalize(x_flat.astype(jnp.float32))
    a_norm = _normalize(anchors.astype(jnp.float32))
    resonance_scores = x_norm @ a_norm.T
    topk_scores, topk_idx = jax.lax.top_k(resonance_scores, K)
    gating = jax.nn.softmax(topk_scores, axis=-1).astype(x.dtype)

    w = jnp.sum(
        jnp.where(
            topk_idx[:, None, :] == jnp.arange(E)[None, :, None],
            gating[:, None, :],
            jnp.zeros_like(gating)[:, None, :],
        ),
        axis=-1,
    )

    BT = 256
    BF = 2048
    grid = (T // BT, E, DFF // BF)
    out = pl.pallas_call(
        _ffn_kernel,
        grid=grid,
        in_specs=[
            pl.BlockSpec((BT, Dq), lambda t, e, f: (t, 0)),
            pl.BlockSpec((BT, E), lambda t, e, f: (t, 0)),
            pl.BlockSpec((1, Dq, BF), lambda t, e, f: (e, 0, f)),
            pl.BlockSpec((1, 1, BF), lambda t, e, f: (e, 0, f)),
            pl.BlockSpec((1, BF, Dq), lambda t, e, f: (e, f, 0)),
            pl.BlockSpec((1, 1, Dq), lambda t, e, f: (e, 0, 0)),
        ],
        out_specs=pl.BlockSpec((BT, Dq), lambda t, e, f: (t, 0)),
        out_shape=jax.ShapeDtypeStruct((T, Dq), x.dtype),
        scratch_shapes=[pltpu.VMEM((BT, Dq), jnp.float32)],
        compiler_params=pltpu.CompilerParams(
            dimension_semantics=("parallel", "arbitrary", "arbitrary"),
        ),
    )(x_flat, w, W1, b1.reshape(E, 1, DFF), W2, b2.reshape(E, 1, Dq))

    output = out.reshape(Bq, Sq, Dq)
    return (output, resonance_scores.astype(x.dtype), topk_idx, a_norm.astype(x.dtype))


# speedup vs baseline: 1.1588x; 1.1588x over previous
import jax
import jax.numpy as jnp
from jax.experimental import pallas as pl
from jax.experimental.pallas import tpu as pltpu
from jax.experimental.pallas import tpu_sc as plsc

K = 2
BLK = 256
LANES = 128
WIN = 128


def _normalize(v, eps=1e-08):
    n = jnp.sqrt(jnp.sum(v * v, axis=-1, keepdims=True))
    return v / jnp.maximum(n, eps)


def _ffn_block_kernel(be_ref, nreal_ref, x_ref, w_ref, W1_ref, b1_ref, W2_ref,
                      b2_ref, y_ref):
    b = pl.program_id(0)

    wrow = w_ref[0, 0][:, None]

    @pl.when(b < nreal_ref[0])
    def _():
        xb = x_ref[...].astype(jnp.bfloat16)
        h = jax.nn.gelu(
            jax.lax.dot_general(
                xb, W1_ref[0],
                (((1,), (0,)), ((), ())),
                preferred_element_type=jnp.float32,
            )
            + b1_ref[0, 0]
        )
        oe = jax.lax.dot_general(
            h.astype(jnp.bfloat16), W2_ref[0],
            (((1,), (0,)), ((), ())),
            preferred_element_type=jnp.float32,
        )
        y_ref[...] = (oe + b2_ref[0, 0]) * wrow

    @pl.when(b >= nreal_ref[0])
    def _():
        y_ref[...] = jnp.zeros_like(y_ref)


def _sc_gather_chunks(src2, idx, nchunks):
    mesh = plsc.VectorSubcoreMesh(core_axis_name="core",
                                  subcore_axis_name="subcore")

    @pl.kernel(out_type=jax.ShapeDtypeStruct((nchunks, LANES), src2.dtype),
               mesh=mesh)
    def _k(x_hbm, i_hbm, o_hbm):
        def body(i_vmem, o_vmem):
            pltpu.sync_copy(x_hbm.at[i_vmem.at[0]], o_vmem)

        pltpu.emit_pipeline(
            body,
            grid=(nchunks // WIN,),
            in_specs=[pl.BlockSpec((1, WIN), index_map=lambda i: (0, i))],
            out_specs=[pl.BlockSpec((WIN, LANES), index_map=lambda i: (i, 0))],
            core_axis_name=("core", "subcore"),
            dimension_semantics=(pltpu.PARALLEL,),
        )(i_hbm, o_hbm)

    return _k(src2, idx)


def _sc_combine_chunks(y2, pos0, pos1, nchunks):
    mesh = plsc.VectorSubcoreMesh(core_axis_name="core",
                                  subcore_axis_name="subcore")

    @pl.kernel(
        out_type=jax.ShapeDtypeStruct((nchunks, LANES), y2.dtype),
        mesh=mesh,
        scratch_types=[pltpu.VMEM((WIN, LANES), y2.dtype),
                       pltpu.VMEM((WIN, LANES), y2.dtype)],
    )
    def _k(y_hbm, i0_hbm, i1_hbm, o_hbm, s0, s1):
        def body(i0_vmem, i1_vmem, o_vmem):
            pltpu.sync_copy(y_hbm.at[i0_vmem.at[0]], s0)
            pltpu.sync_copy(y_hbm.at[i1_vmem.at[0]], s1)

            @pl.loop(0, WIN)
            def _(r):
                @pl.loop(0, LANES, step=16)
                def _(c):
                    ds = pl.ds(c, 16)
                    o_vmem.at[r, ds][...] = (s0.at[r, ds][...]
                                             + s1.at[r, ds][...])

        pltpu.emit_pipeline(
            body,
            grid=(nchunks // WIN,),
            in_specs=[pl.BlockSpec((1, WIN), index_map=lambda i: (0, i)),
                      pl.BlockSpec((1, WIN), index_map=lambda i: (0, i))],
            out_specs=[pl.BlockSpec((WIN, LANES), index_map=lambda i: (i, 0))],
            core_axis_name=("core", "subcore"),
            dimension_semantics=(pltpu.PARALLEL,),
        )(i0_hbm, i1_hbm, o_hbm)

    return _k(y2, pos0, pos1)


def kernel(x, anchors, W1, b1, W2, b2):
    Bq, Sq, Dq = x.shape
    E, _, DFF = W1.shape
    T = Bq * Sq
    TK = T * K
    x_flat = x.reshape(T, Dq)

    x_norm = _normalize(x_flat.astype(jnp.float32))
    a_norm = _normalize(anchors.astype(jnp.float32))
    resonance_scores = x_norm @ a_norm.T
    topk_scores, topk_idx = jax.lax.top_k(resonance_scores, K)
    gating = jax.nn.softmax(topk_scores, axis=-1).astype(x.dtype)

    ef = topk_idx.reshape(TK)
    gf = gating.reshape(TK)
    oh = (ef[:, None] == jnp.arange(E)[None, :]).astype(jnp.int32)
    cum = jnp.cumsum(oh, axis=0)
    counts = cum[-1]
    rank = jnp.take_along_axis(cum, ef[:, None], axis=1)[:, 0] - 1
    nb = (counts + BLK - 1) // BLK
    nb_cum = jnp.cumsum(nb)
    nreal = nb_cum[-1]
    pad_off = (nb_cum - nb) * BLK
    pos = (pad_off[ef] + rank).astype(jnp.int32)

    NB = TK // BLK + E
    P = NB * BLK
    tok_padded = jnp.zeros((P,), jnp.int32).at[pos].set(
        (jnp.arange(TK, dtype=jnp.int32) // K))
    w_padded = jnp.zeros((P,), jnp.float32).at[pos].set(gf)
    bidx = jnp.arange(NB, dtype=jnp.int32)
    be = jnp.searchsorted(nb_cum, bidx, side="right").astype(jnp.int32)
    last = be[jnp.maximum(nreal - 1, 0)]
    be = jnp.where(bidx < nreal, jnp.minimum(be, E - 1), last)
    C = Dq // LANES
    cj = jnp.arange(C, dtype=jnp.int32)[None, :]
    posTK = pos.reshape(T, K)
    pos0 = (posTK[:, 0:1] * C + cj).reshape(1, T * C)
    pos1 = (posTK[:, 1:2] * C + cj).reshape(1, T * C)
    tok_chunks = (tok_padded[:, None] * C + cj).reshape(1, P * C)

    x_padded = _sc_gather_chunks(x_flat.reshape(T * C, LANES), tok_chunks,
                                 P * C).reshape(P, Dq)

    W1b = W1.astype(jnp.bfloat16)
    W2b = W2.astype(jnp.bfloat16)
    grid_spec = pltpu.PrefetchScalarGridSpec(
        num_scalar_prefetch=2,
        grid=(NB,),
        in_specs=[
            pl.BlockSpec((BLK, Dq), lambda b, be_r, nr: (b, 0)),
            pl.BlockSpec((1, 1, BLK), lambda b, be_r, nr: (b, 0, 0)),
            pl.BlockSpec((1, Dq, DFF), lambda b, be_r, nr: (be_r[b], 0, 0)),
            pl.BlockSpec((1, 1, DFF), lambda b, be_r, nr: (be_r[b], 0, 0)),
            pl.BlockSpec((1, DFF, Dq), lambda b, be_r, nr: (be_r[b], 0, 0)),
            pl.BlockSpec((1, 1, Dq), lambda b, be_r, nr: (be_r[b], 0, 0)),
        ],
        out_specs=pl.BlockSpec((BLK, Dq), lambda b, be_r, nr: (b, 0)),
    )
    y_padded = pl.pallas_call(
        _ffn_block_kernel,
        grid_spec=grid_spec,
        out_shape=jax.ShapeDtypeStruct((P, Dq), jnp.float32),
        compiler_params=pltpu.CompilerParams(
            dimension_semantics=("arbitrary",),
        ),
    )(be, nreal.reshape(1), x_padded, w_padded.reshape(NB, 1, BLK), W1b,
      b1.reshape(E, 1, DFF), W2b, b2.reshape(E, 1, Dq))

    out = _sc_combine_chunks(y_padded.reshape(P * C, LANES), pos0, pos1, T * C)

    output = out.reshape(Bq, Sq, Dq).astype(x.dtype)
    return (output, resonance_scores.astype(x.dtype), topk_idx,
            a_norm.astype(x.dtype))
